# baseline (speedup 1.0000x reference)
import functools

import jax
import jax.numpy as jnp
from jax import lax
from jax.experimental import pallas as pl
from jax.experimental.pallas import tpu as pltpu
from jax.experimental.pallas import tpu_sc as plsc

B = 16384
E = 16
NC = 2
NS = 16
NW = NC * NS
BPW = B // NW
CH = 128
NCH = BPW // CH


def _gmf_body(user_hbm, item_hbm, ut_hbm, it_hbm, out_hbm,
              uidx_v, iidx_v, ubuf_v, ibuf_v, out_v, sems):
    wid = lax.axis_index("s") * NC + lax.axis_index("c")
    base = wid * BPW

    for c in range(NCH):
        pltpu.sync_copy(user_hbm.at[pl.ds(base + c * CH, CH)], uidx_v.at[c])
        pltpu.sync_copy(item_hbm.at[pl.ds(base + c * CH, CH)], iidx_v.at[c])

    copies = []
    for d in range(E):
        for c in range(NCH):
            copies.append(pltpu.async_copy(
                ut_hbm.at[d].at[uidx_v.at[c]], ubuf_v.at[d, c], sems.at[0]))
            copies.append(pltpu.async_copy(
                it_hbm.at[d].at[iidx_v.at[c]], ibuf_v.at[d, c], sems.at[1]))
    for cp in copies:
        cp.wait()

    for c in range(NCH):
        for g in range(CH // E):
            acc = jnp.zeros((E,), jnp.float32)
            for d in range(E):
                u = ubuf_v.at[d, c][pl.ds(g * E, E)]
                v = ibuf_v.at[d, c][pl.ds(g * E, E)]
                acc = acc + u * v
            out_v[pl.ds(c * CH + g * E, E)] = acc

    pltpu.sync_copy(out_v, out_hbm.at[pl.ds(base, BPW)])


_gmf = functools.partial(
    pl.kernel,
    mesh=plsc.VectorSubcoreMesh(core_axis_name="c", subcore_axis_name="s"),
    out_type=jax.ShapeDtypeStruct((B,), jnp.float32),
    scratch_types=[
        pltpu.VMEM((NCH, CH), jnp.int32),
        pltpu.VMEM((NCH, CH), jnp.int32),
        pltpu.VMEM((E, NCH, CH), jnp.float32),
        pltpu.VMEM((E, NCH, CH), jnp.float32),
        pltpu.VMEM((BPW,), jnp.float32),
        pltpu.SemaphoreType.DMA((2,)),
    ],
    compiler_params=pltpu.CompilerParams(
        needs_layout_passes=False, use_tc_tiling_on_sc=False
    ),
)(_gmf_body)


def kernel(user, item, embed_user_GMF, embed_item_GMF):
    user = user.astype(jnp.int32)
    item = item.astype(jnp.int32)
    return _gmf(user, item, embed_user_GMF.T, embed_item_GMF.T)


# speedup vs baseline: 3.1873x; 3.1873x over previous
import functools

import jax
import jax.numpy as jnp
from jax import lax
from jax.experimental import pallas as pl
from jax.experimental.pallas import tpu as pltpu
from jax.experimental.pallas import tpu_sc as plsc

B = 16384
E = 16
NC = 2
NS = 16
NW = NC * NS
BPW = B // NW
CH = 128
NCH = BPW // CH
GRP = BPW // E


def _gmf_body(user_hbm, item_hbm, ut_hbm, it_hbm, out_hbm,
              uidx_v, iidx_v, urows_v, irows_v, out_v, sem):
    wid = lax.axis_index("s") * NC + lax.axis_index("c")
    base = wid * BPW

    for c in range(NCH):
        pltpu.sync_copy(user_hbm.at[pl.ds(base + c * CH, CH)], uidx_v.at[c])
        pltpu.sync_copy(item_hbm.at[pl.ds(base + c * CH, CH)], iidx_v.at[c])

    copies = []
    for c in range(NCH):
        copies.append(pltpu.async_copy(ut_hbm.at[uidx_v.at[c]], urows_v.at[c], sem))
        copies.append(pltpu.async_copy(it_hbm.at[iidx_v.at[c]], irows_v.at[c], sem))
    for cp in copies:
        cp.wait()

    lane = lax.iota(jnp.int32, 16)

    def group(g, carry):
        c = g // (CH // E)
        r0 = (g % (CH // E)) * E
        ridx = r0 + lane
        cvec = jnp.full((16,), 0, jnp.int32) + c
        acc = jnp.zeros((16,), jnp.float32)
        for d in range(E):
            dvec = jnp.full((16,), d, jnp.int32)
            u = plsc.load_gather(urows_v, [cvec, ridx, dvec])
            v = plsc.load_gather(irows_v, [cvec, ridx, dvec])
            acc = acc + u * v
        out_v[pl.ds(g * E, E)] = acc
        return carry

    lax.fori_loop(0, GRP, group, 0)

    pltpu.sync_copy(out_v, out_hbm.at[pl.ds(base, BPW)])


_gmf = functools.partial(
    pl.kernel,
    mesh=plsc.VectorSubcoreMesh(core_axis_name="c", subcore_axis_name="s"),
    out_type=jax.ShapeDtypeStruct((B,), jnp.float32),
    scratch_types=[
        pltpu.VMEM((NCH, CH), jnp.int32),
        pltpu.VMEM((NCH, CH), jnp.int32),
        pltpu.VMEM((NCH, CH, E), jnp.float32),
        pltpu.VMEM((NCH, CH, E), jnp.float32),
        pltpu.VMEM((BPW,), jnp.float32),
        pltpu.SemaphoreType.DMA,
    ],
    compiler_params=pltpu.CompilerParams(
        needs_layout_passes=False, use_tc_tiling_on_sc=False
    ),
)(_gmf_body)


def kernel(user, item, embed_user_GMF, embed_item_GMF):
    user = user.astype(jnp.int32)
    item = item.astype(jnp.int32)
    return _gmf(user, item, embed_user_GMF, embed_item_GMF)


# speedup vs baseline: 4.6607x; 1.4623x over previous
import functools

import jax
import jax.numpy as jnp
from jax import lax
from jax.experimental import pallas as pl
from jax.experimental.pallas import tpu as pltpu
from jax.experimental.pallas import tpu_sc as plsc

B = 16384
E = 16
V = 1000000
NC = 2
NS = 16
NW = NC * NS
BPW = B // NW
CH = 64
NCH = BPW // CH
GPC = CH // E

NB = 8192
NBLK = (V + NB - 1) // NB


def _repack_body(t_ref, o_ref):
    x = t_ref[...]
    y = x.T.reshape(NB // 8, 8, E)
    for s in range(8):
        o_ref[:, s * E:(s + 1) * E] = y[:, s, :]


_repack = pl.pallas_call(
    _repack_body,
    grid=(NBLK,),
    in_specs=[pl.BlockSpec((E, NB), lambda g: (0, g))],
    out_specs=pl.BlockSpec((NB // 8, 128), lambda g: (g, 0)),
    out_shape=jax.ShapeDtypeStruct((V * E // 128, 128), jnp.float32),
)


def _gmf_body(user_hbm, item_hbm, ut_hbm, it_hbm, out_hbm,
              uidx_v, iidx_v, udiv_v, idiv_v, ubuf_v, ibuf_v, out_v, sems):
    wid = lax.axis_index("s") * NC + lax.axis_index("c")
    base = wid * BPW

    pltpu.sync_copy(user_hbm.at[pl.ds(base, BPW)], uidx_v)
    pltpu.sync_copy(item_hbm.at[pl.ds(base, BPW)], iidx_v)

    for j in range(BPW // E):
        c, r = j // (CH // E), (j % (CH // E)) * E
        udiv_v.at[c][pl.ds(r, E)] = uidx_v[pl.ds(j * E, E)] >> 3
        idiv_v.at[c][pl.ds(r, E)] = iidx_v[pl.ds(j * E, E)] >> 3

    lane = lax.iota(jnp.int32, 16)

    def fire(c):
        p = c % 2
        return (
            pltpu.async_copy(ut_hbm.at[udiv_v.at[c]], ubuf_v.at[p], sems.at[p, 0]),
            pltpu.async_copy(it_hbm.at[idiv_v.at[c]], ibuf_v.at[p], sems.at[p, 1]),
        )

    inflight = {0: fire(0)}
    for c in range(NCH):
        if c + 1 < NCH:
            inflight[c + 1] = fire(c + 1)
        for cp in inflight.pop(c):
            cp.wait()
        p = c % 2
        for g in range(GPC):
            uvec = uidx_v[pl.ds(c * CH + g * E, E)]
            ivec = iidx_v[pl.ds(c * CH + g * E, E)]
            ucol0 = (uvec & 7) * E
            icol0 = (ivec & 7) * E
            row = g * E + lane
            acc = jnp.zeros((E,), jnp.float32)
            for d in range(E):
                u = plsc.load_gather(ubuf_v, [jnp.full((E,), p, jnp.int32), row, ucol0 + d])
                v = plsc.load_gather(ibuf_v, [jnp.full((E,), p, jnp.int32), row, icol0 + d])
                acc = acc + u * v
            out_v[pl.ds(c * CH + g * E, E)] = acc

    pltpu.sync_copy(out_v, out_hbm.at[pl.ds(base, BPW)])


_gmf = functools.partial(
    pl.kernel,
    mesh=plsc.VectorSubcoreMesh(core_axis_name="c", subcore_axis_name="s"),
    out_type=jax.ShapeDtypeStruct((B,), jnp.float32),
    scratch_types=[
        pltpu.VMEM((BPW,), jnp.int32),
        pltpu.VMEM((BPW,), jnp.int32),
        pltpu.VMEM((NCH, CH), jnp.int32),
        pltpu.VMEM((NCH, CH), jnp.int32),
        pltpu.VMEM((2, CH, 128), jnp.float32),
        pltpu.VMEM((2, CH, 128), jnp.float32),
        pltpu.VMEM((BPW,), jnp.float32),
        pltpu.SemaphoreType.DMA((2, 2)),
    ],
    compiler_params=pltpu.CompilerParams(needs_layout_passes=False),
)(_gmf_body)


def kernel(user, item, embed_user_GMF, embed_item_GMF):
    user = user.astype(jnp.int32)
    item = item.astype(jnp.int32)
    ut = _repack(embed_user_GMF.T)
    it = _repack(embed_item_GMF.T)
    return _gmf(user, item, ut, it)
